# baseline (device time: 13976 ns/iter reference)
import jax
import jax.numpy as jnp
from jax import lax
from jax.experimental import pallas as pl
from jax.experimental.pallas import tpu as pltpu

C = 4


def kernel(partial, resid, gamma):
    _, M, D = partial.shape
    MB = M // 2
    CH = MB // C

    def body(partial_ref, resid_ref, gamma_ref, out_ref,
             ysend, yrecv, xsend, xrecv,
             ysend_sems, yrecv_sems, xsend_sems, xrecv_sems):
        my_x = lax.axis_index("x")
        my_y = lax.axis_index("y")
        ynbr = (my_x, 1 - my_y)
        xnbr = (1 - my_x, my_y)

        barrier_sem = pltpu.get_barrier_semaphore()
        for nbr in (ynbr, xnbr):
            pl.semaphore_signal(barrier_sem, inc=1, device_id=nbr,
                                device_id_type=pl.DeviceIdType.MESH)
        pl.semaphore_wait(barrier_sem, 2)

        blk = my_x * MB
        oblk = (1 - my_x) * MB

        y_rdmas = []
        for c in range(C):
            ysend[pl.ds(c * CH, CH), :] = (
                partial_ref[0, pl.ds(blk + c * CH, CH), :]
                .astype(jnp.bfloat16))
            r = pltpu.make_async_remote_copy(
                src_ref=ysend.at[pl.ds(c * CH, CH)],
                dst_ref=yrecv.at[pl.ds(c * CH, CH)],
                send_sem=ysend_sems.at[c],
                recv_sem=yrecv_sems.at[c],
                device_id=ynbr,
                device_id_type=pl.DeviceIdType.MESH,
            )
            r.start()
            y_rdmas.append(r)

        x_rdmas = []
        for c in range(C):
            y_rdmas[c].wait_recv()
            yv = (partial_ref[0, pl.ds(blk + c * CH, CH), :]
                  + yrecv[pl.ds(c * CH, CH), :].astype(jnp.float32)
                  + resid_ref[pl.ds(blk + c * CH, CH), :])
            rms = jnp.sqrt(jnp.mean(yv * yv, axis=-1, keepdims=True) + 1e-6)
            o = yv / rms * gamma_ref[...]
            out_ref[pl.ds(blk + c * CH, CH), :] = o
            xsend[pl.ds(c * CH, CH), :] = o.astype(jnp.bfloat16)
            r = pltpu.make_async_remote_copy(
                src_ref=xsend.at[pl.ds(c * CH, CH)],
                dst_ref=xrecv.at[pl.ds(c * CH, CH)],
                send_sem=xsend_sems.at[c],
                recv_sem=xrecv_sems.at[c],
                device_id=xnbr,
                device_id_type=pl.DeviceIdType.MESH,
            )
            r.start()
            x_rdmas.append(r)

        for c in range(C):
            x_rdmas[c].wait_recv()
            out_ref[pl.ds(oblk + c * CH, CH), :] = (
                xrecv[pl.ds(c * CH, CH), :].astype(jnp.float32))

        for c in range(C):
            y_rdmas[c].wait_send()
            x_rdmas[c].wait_send()

    return pl.pallas_call(
        body,
        out_shape=jax.ShapeDtypeStruct((M, D), jnp.float32),
        in_specs=[pl.BlockSpec(memory_space=pltpu.VMEM)] * 3,
        out_specs=pl.BlockSpec(memory_space=pltpu.VMEM),
        scratch_shapes=[
            pltpu.VMEM((MB, D), jnp.bfloat16),
            pltpu.VMEM((MB, D), jnp.bfloat16),
            pltpu.VMEM((MB, D), jnp.bfloat16),
            pltpu.VMEM((MB, D), jnp.bfloat16),
            pltpu.SemaphoreType.DMA((C,)),
            pltpu.SemaphoreType.DMA((C,)),
            pltpu.SemaphoreType.DMA((C,)),
            pltpu.SemaphoreType.DMA((C,)),
        ],
        compiler_params=pltpu.CompilerParams(collective_id=0),
    )(partial, resid, gamma.reshape(1, D))
